# baseline (speedup 1.0000x reference)
import functools

import jax
import jax.numpy as jnp
from jax import lax
from jax.experimental import pallas as pl
from jax.experimental.pallas import tpu as pltpu
from jax.experimental.pallas import tpu_sc as plsc

_BINS = (-1.1503, -0.6745, -0.3186, 0.0, 0.3186, 0.6745, 1.1503)

_N = 32768
_STRIDE = 1024
_LANES = 16
_IDX_CHUNK = 128


def _sc_kernel(n_workers: int, bpw: int, x_hbm, out_hbm, idx_v, vals_v,
               res_v, sem):
    nc = plsc.get_sparse_core_info().num_cores
    wid = lax.axis_index("s") * nc + lax.axis_index("c")
    base = pl.multiple_of(wid * bpw, bpw)

    lanes = lax.iota(jnp.int32, _LANES)
    n_chunks = bpw // _LANES
    per_row = _IDX_CHUNK // _LANES

    for c in range(n_chunks):
        row, col = c // per_row, (c % per_row) * _LANES
        idx_v[row, pl.ds(col, _LANES)] = (base + c * _LANES + lanes) * _STRIDE

    n_gathers = bpw // _IDX_CHUNK
    copies = [
        pltpu.async_copy(x_hbm.at[idx_v.at[j]],
                         vals_v.at[pl.ds(j * _IDX_CHUNK, _IDX_CHUNK)], sem)
        for j in range(n_gathers)
    ]
    for cp in copies:
        cp.wait()

    for c in range(n_chunks):
        v = vals_v[pl.ds(c * _LANES, _LANES)]
        acc = jnp.maximum(jnp.sign(v - _BINS[0]), 0.0)
        for b in _BINS[1:]:
            acc = acc + jnp.maximum(jnp.sign(v - b), 0.0)
        res_v[pl.ds(c * _LANES, _LANES)] = acc.astype(jnp.int32)

    pltpu.sync_copy(res_v, out_hbm.at[pl.ds(base, bpw)])


def kernel(x):
    info = plsc.get_sparse_core_info()
    n_workers = info.num_cores * info.num_subcores
    bpw = _N // n_workers

    mesh = plsc.VectorSubcoreMesh(core_axis_name="c", subcore_axis_name="s")
    k = functools.partial(
        pl.kernel,
        mesh=mesh,
        out_type=jax.ShapeDtypeStruct((_N,), jnp.int32),
        scratch_types=[
            pltpu.VMEM((bpw // _IDX_CHUNK, _IDX_CHUNK), jnp.int32),
            pltpu.VMEM((bpw,), jnp.float32),
            pltpu.VMEM((bpw,), jnp.int32),
            pltpu.SemaphoreType.DMA,
        ],
    )(functools.partial(_sc_kernel, n_workers, bpw))

    return k(x.reshape(-1))


# speedup vs baseline: 1.0019x; 1.0019x over previous
import functools

import jax
import jax.numpy as jnp
from jax import lax
from jax.experimental import pallas as pl
from jax.experimental.pallas import tpu as pltpu
from jax.experimental.pallas import tpu_sc as plsc

_BINS = (-1.1503, -0.6745, -0.3186, 0.0, 0.3186, 0.6745, 1.1503)

_N = 32768
_STRIDE = 1024
_LANES = 16
_IDX_CHUNK = 128


def _sc_kernel(bpw: int, x_hbm, out_hbm, idx_v, vals_v, res_v, sem):
    nc = plsc.get_sparse_core_info().num_cores
    wid = lax.axis_index("s") * nc + lax.axis_index("c")
    base = pl.multiple_of(wid * bpw, bpw)

    lanes = lax.iota(jnp.int32, _LANES)
    per_row = _IDX_CHUNK // _LANES
    n_gathers = bpw // _IDX_CHUNK

    copies = []
    for j in range(n_gathers):
        for r in range(per_row):
            k0 = j * _IDX_CHUNK + r * _LANES
            idx_v[j, pl.ds(r * _LANES, _LANES)] = (base + k0 + lanes) * _STRIDE
        copies.append(
            pltpu.async_copy(x_hbm.at[idx_v.at[j]],
                             vals_v.at[pl.ds(j * _IDX_CHUNK, _IDX_CHUNK)],
                             sem))
    for cp in copies:
        cp.wait()

    for c in range(bpw // _LANES):
        v = vals_v[pl.ds(c * _LANES, _LANES)]
        acc = jnp.maximum(jnp.sign(v - _BINS[0]), 0.0)
        for b in _BINS[1:]:
            acc = acc + jnp.maximum(jnp.sign(v - b), 0.0)
        res_v[pl.ds(c * _LANES, _LANES)] = acc.astype(jnp.int32)

    pltpu.sync_copy(res_v, out_hbm.at[pl.ds(base, bpw)])


def kernel(x):
    info = plsc.get_sparse_core_info()
    n_workers = info.num_cores * info.num_subcores
    bpw = _N // n_workers

    mesh = plsc.VectorSubcoreMesh(core_axis_name="c", subcore_axis_name="s")
    k = functools.partial(
        pl.kernel,
        mesh=mesh,
        out_type=jax.ShapeDtypeStruct((_N,), jnp.int32),
        scratch_types=[
            pltpu.VMEM((bpw // _IDX_CHUNK, _IDX_CHUNK), jnp.int32),
            pltpu.VMEM((bpw,), jnp.float32),
            pltpu.VMEM((bpw,), jnp.int32),
            pltpu.SemaphoreType.DMA,
        ],
    )(functools.partial(_sc_kernel, bpw))

    return k(x.reshape(-1))


# speedup vs baseline: 5.8566x; 5.8454x over previous
import functools

import jax
import jax.numpy as jnp
from jax import lax
from jax.experimental import pallas as pl
from jax.experimental.pallas import tpu as pltpu
from jax.experimental.pallas import tpu_sc as plsc

_BINS = (-1.1503, -0.6745, -0.3186, 0.0, 0.3186, 0.6745, 1.1503)

_N = 32768
_LANES = 16
_SUB = 8
_LD = 1024
_ROUND = 64
_RELEM = _ROUND * _SUB
_TRASH = _RELEM


def _bucketize(v):
    acc = jnp.maximum(jnp.sign(v - _BINS[0]), 0.0)
    for b in _BINS[1:]:
        acc = acc + jnp.maximum(jnp.sign(v - b), 0.0)
    return acc.astype(jnp.int32)


def _sc_kernel(bpw: int, x_hbm, out_hbm, idx_v, tile_v, vals_v, res_v, sem):
    nc = plsc.get_sparse_core_info().num_cores
    wid = lax.axis_index("s") * nc + lax.axis_index("c")
    base = pl.multiple_of(wid * bpw, bpw)
    blk0 = pl.multiple_of(wid * (bpw // _SUB), bpw // _SUB)

    lanes = lax.iota(jnp.int32, _LANES)
    onehot0 = jnp.maximum(1 - lanes, 0)
    n_blocks = bpw // _SUB

    for r in range(n_blocks // _LANES):
        idx_v[pl.ds(r * _LANES, _LANES)] = blk0 + r * _LANES + lanes

    for rnd in range(n_blocks // _ROUND):
        pltpu.async_copy(
            x_hbm.at[idx_v.at[pl.ds(rnd * _ROUND, _ROUND)], :, pl.ds(0, 128)],
            tile_v, sem).wait()

        def compact(b, carry):
            for s in range(_SUB):
                v = tile_v[b, s, pl.ds(0, _LANES)]
                pos = b * _SUB + s
                plsc.store_scatter(
                    vals_v, [(pos - _TRASH) * onehot0 + _TRASH], v)
            return carry
        lax.fori_loop(0, _ROUND, compact, 0)

        for g in range(_RELEM // _LANES):
            vv = vals_v[pl.ds(g * _LANES, _LANES)]
            res_v[pl.ds(rnd * _RELEM + g * _LANES, _LANES)] = _bucketize(vv)

    pltpu.sync_copy(res_v, out_hbm.at[pl.ds(base, bpw)])


def kernel(x):
    info = plsc.get_sparse_core_info()
    n_workers = info.num_cores * info.num_subcores
    bpw = _N // n_workers

    x3 = x.reshape(_N * 2 // _SUB, _SUB, _LD)

    mesh = plsc.VectorSubcoreMesh(core_axis_name="c", subcore_axis_name="s")
    k = functools.partial(
        pl.kernel,
        mesh=mesh,
        compiler_params=pltpu.CompilerParams(needs_layout_passes=False),
        out_type=jax.ShapeDtypeStruct((_N,), jnp.int32),
        scratch_types=[
            pltpu.VMEM((bpw // _SUB,), jnp.int32),
            pltpu.VMEM((_ROUND, _SUB, 128), jnp.float32),
            pltpu.VMEM((_RELEM + _LANES,), jnp.float32),
            pltpu.VMEM((bpw,), jnp.int32),
            pltpu.SemaphoreType.DMA,
        ],
    )(functools.partial(_sc_kernel, bpw))

    return k(x3)


# speedup vs baseline: 7.1001x; 1.2123x over previous
import functools

import jax
import jax.numpy as jnp
from jax import lax
from jax.experimental import pallas as pl
from jax.experimental.pallas import tpu as pltpu
from jax.experimental.pallas import tpu_sc as plsc

_BINS = (-1.1503, -0.6745, -0.3186, 0.0, 0.3186, 0.6745, 1.1503)

_N = 32768
_LANES = 16
_SUB = 8
_LD = 1024
_ROUND = 32
_RELEM = _ROUND * _SUB


def _bucketize(v):
    acc = (v > _BINS[0]).astype(jnp.int32)
    for b in _BINS[1:]:
        acc = acc + (v > b).astype(jnp.int32)
    return acc


def _sc_kernel(bpw: int, x_hbm, out_hbm, idx_v, buf0, buf1, res_v,
               sem0, sem1):
    nc = plsc.get_sparse_core_info().num_cores
    wid = lax.axis_index("s") * nc + lax.axis_index("c")
    base = pl.multiple_of(wid * bpw, bpw)
    blk0 = pl.multiple_of(wid * (bpw // _SUB), bpw // _SUB)

    lanes = lax.iota(jnp.int32, _LANES)
    zeros = lanes * 0
    n_blocks = bpw // _SUB

    for r in range(n_blocks // _LANES):
        idx_v[pl.ds(r * _LANES, _LANES)] = blk0 + r * _LANES + lanes

    bufs = (buf0, buf1)
    sems = (sem0, sem1)

    def fire(rnd):
        return pltpu.async_copy(
            x_hbm.at[idx_v.at[pl.ds(rnd * _ROUND, _ROUND)], :, pl.ds(0, 128)],
            bufs[rnd % 2], sems[rnd % 2])

    n_rounds = n_blocks // _ROUND
    cps = [fire(0), None]
    for rnd in range(n_rounds):
        if rnd + 1 < n_rounds:
            cps[(rnd + 1) % 2] = fire(rnd + 1)
        cps[rnd % 2].wait()
        buf = bufs[rnd % 2]
        for g in range(_RELEM // _LANES):
            e = g * _LANES + lanes
            v = plsc.load_gather(
                buf, [lax.shift_right_logical(e, 3),
                      lax.bitwise_and(e, _SUB - 1), zeros])
            res_v[pl.ds(rnd * _RELEM + g * _LANES, _LANES)] = _bucketize(v)

    pltpu.sync_copy(res_v, out_hbm.at[pl.ds(base, bpw)])


def kernel(x):
    info = plsc.get_sparse_core_info()
    n_workers = info.num_cores * info.num_subcores
    bpw = _N // n_workers

    x3 = x.reshape(_N * 2 // _SUB, _SUB, _LD)

    mesh = plsc.VectorSubcoreMesh(core_axis_name="c", subcore_axis_name="s")
    k = functools.partial(
        pl.kernel,
        mesh=mesh,
        compiler_params=pltpu.CompilerParams(needs_layout_passes=False),
        out_type=jax.ShapeDtypeStruct((_N,), jnp.int32),
        scratch_types=[
            pltpu.VMEM((bpw // _SUB,), jnp.int32),
            pltpu.VMEM((_ROUND, _SUB, 128), jnp.float32),
            pltpu.VMEM((_ROUND, _SUB, 128), jnp.float32),
            pltpu.VMEM((bpw,), jnp.int32),
            pltpu.SemaphoreType.DMA,
            pltpu.SemaphoreType.DMA,
        ],
    )(functools.partial(_sc_kernel, bpw))

    return k(x3)


# speedup vs baseline: 7.3696x; 1.0380x over previous
import functools

import jax
import jax.numpy as jnp
from jax import lax
from jax.experimental import pallas as pl
from jax.experimental.pallas import tpu as pltpu
from jax.experimental.pallas import tpu_sc as plsc

_BINS = (-1.1503, -0.6745, -0.3186, 0.0, 0.3186, 0.6745, 1.1503)

_N = 32768
_LANES = 16
_SUB = 8
_LD = 1024
_ROUND = 32
_RELEM = _ROUND * _SUB


def _bucketize(v):
    acc = (v > _BINS[0]).astype(jnp.int32)
    for b in _BINS[1:]:
        acc = acc + (v > b).astype(jnp.int32)
    return acc


def _sc_kernel(bpw: int, x_hbm, out_hbm, buf0, buf1, res_v,
               sem0, sem1):
    nc = plsc.get_sparse_core_info().num_cores
    wid = lax.axis_index("s") * nc + lax.axis_index("c")
    base = pl.multiple_of(wid * bpw, bpw)
    blk0 = pl.multiple_of(wid * (bpw // _SUB), bpw // _SUB)

    lanes = lax.iota(jnp.int32, _LANES)
    zeros = lanes * 0
    n_blocks = bpw // _SUB

    bufs = (buf0, buf1)
    sems = (sem0, sem1)

    def fire(rnd):
        return pltpu.async_copy(
            x_hbm.at[pl.ds(blk0 + rnd * _ROUND, _ROUND), :, pl.ds(0, 128)],
            bufs[rnd % 2], sems[rnd % 2])

    n_rounds = n_blocks // _ROUND
    cps = [fire(0), None]
    for rnd in range(n_rounds):
        if rnd + 1 < n_rounds:
            cps[(rnd + 1) % 2] = fire(rnd + 1)
        cps[rnd % 2].wait()
        buf = bufs[rnd % 2]
        for g in range(_RELEM // _LANES):
            e = g * _LANES + lanes
            v = plsc.load_gather(
                buf, [lax.shift_right_logical(e, 3),
                      lax.bitwise_and(e, _SUB - 1), zeros])
            res_v[pl.ds(rnd * _RELEM + g * _LANES, _LANES)] = _bucketize(v)

    pltpu.sync_copy(res_v, out_hbm.at[pl.ds(base, bpw)])


def kernel(x):
    info = plsc.get_sparse_core_info()
    n_workers = info.num_cores * info.num_subcores
    bpw = _N // n_workers

    x3 = x.reshape(_N * 2 // _SUB, _SUB, _LD)

    mesh = plsc.VectorSubcoreMesh(core_axis_name="c", subcore_axis_name="s")
    k = functools.partial(
        pl.kernel,
        mesh=mesh,
        compiler_params=pltpu.CompilerParams(needs_layout_passes=False),
        out_type=jax.ShapeDtypeStruct((_N,), jnp.int32),
        scratch_types=[
            pltpu.VMEM((_ROUND, _SUB, 128), jnp.float32),
            pltpu.VMEM((_ROUND, _SUB, 128), jnp.float32),
            pltpu.VMEM((bpw,), jnp.int32),
            pltpu.SemaphoreType.DMA,
            pltpu.SemaphoreType.DMA,
        ],
    )(functools.partial(_sc_kernel, bpw))

    return k(x3)
